# baseline (speedup 1.0000x reference)
import jax
import jax.numpy as jnp
from jax.experimental import pallas as pl

B, T, H, ENT = 4096, 50, 128, 11
BB = 128
G = B // BB


def _dg(a, b, dims):
    return jax.lax.dot_general(a, b, (dims, ((), ())),
                               preferred_element_type=jnp.float32)


def _attack_kernel(stim_ref, tf_ref, st_ref, went_ref, bent_ref, wsty_ref,
                   wkey_ref, scores_ref, logits_ref, atn_ref, arg_ref):
    stim = stim_ref[...]
    wsty = wsty_ref[...]

    logits = _dg(stim, wsty, ((1,), (0,)))
    k = _dg(stim, wkey_ref[...], ((1,), (0,)))
    k1 = k[:, :H]
    k2 = k[:, H:]

    logits_t = _dg(wsty, stim, ((0,), (1,)))
    iota3 = jax.lax.broadcasted_iota(jnp.int32, (3, BB), 0)
    m3 = jnp.max(logits_t, axis=0, keepdims=True)
    atn_t = jnp.min(jnp.where(logits_t >= m3, iota3, 3), axis=0,
                    keepdims=True)
    s1_all = _dg(st_ref[...], k1, ((1,), (1,)))
    s1 = jnp.sum(jnp.where(iota3 == atn_t, s1_all, 0.0), axis=0,
                 keepdims=True)

    targ = _dg(tf_ref[...], went_ref[...], ((1,), (0,))) + bent_ref[...]
    cross = _dg(targ, k2, ((1,), (1,)))
    cross = (cross + s1) * jnp.float32(1.0 / 16.0)

    r_iota = jax.lax.broadcasted_iota(jnp.int32, (BB * T, BB), 0)
    c_iota = jax.lax.broadcasted_iota(jnp.int32, (BB * T, BB), 1)
    mask = (r_iota // T) == c_iota
    scores_ref[...] = jnp.sum(jnp.where(mask, cross, 0.0), axis=1,
                              keepdims=True)

    neg = jnp.where(mask, cross, -jnp.inf)
    cmax = jnp.max(neg, axis=0, keepdims=True)
    t_of_row = r_iota - (r_iota // T) * T
    arg_t = jnp.min(jnp.where(neg >= cmax, t_of_row, T), axis=0,
                    keepdims=True)

    logits_ref[...] = logits
    atn_ref[...] = atn_t.reshape(1, 1, BB)
    arg_ref[...] = arg_t.reshape(1, 1, BB)


def kernel(stim, targFeats, styleTable, W_ent, b_ent, W_style, W_key):
    tf_flat = targFeats.reshape(B * T, ENT)
    bent2 = b_ent.reshape(1, H)
    full = lambda i: (0, 0)
    row = lambda i: (i, 0)
    scores, logits, atn, arg = pl.pallas_call(
        _attack_kernel,
        grid=(G,),
        in_specs=[
            pl.BlockSpec((BB, 2 * H), row),
            pl.BlockSpec((BB * T, ENT), row),
            pl.BlockSpec((3, H), full),
            pl.BlockSpec((ENT, H), full),
            pl.BlockSpec((1, H), full),
            pl.BlockSpec((2 * H, 3), full),
            pl.BlockSpec((2 * H, 2 * H), full),
        ],
        out_specs=[
            pl.BlockSpec((BB * T, 1), row),
            pl.BlockSpec((BB, 3), row),
            pl.BlockSpec((1, 1, BB), lambda i: (i, 0, 0)),
            pl.BlockSpec((1, 1, BB), lambda i: (i, 0, 0)),
        ],
        out_shape=[
            jax.ShapeDtypeStruct((B * T, 1), jnp.float32),
            jax.ShapeDtypeStruct((B, 3), jnp.float32),
            jax.ShapeDtypeStruct((G, 1, BB), jnp.int32),
            jax.ShapeDtypeStruct((G, 1, BB), jnp.int32),
        ],
    )(stim, tf_flat, styleTable, W_ent, bent2, W_style, W_key)
    return (scores.reshape(B, T), logits, atn.reshape(B), arg.reshape(B))
